# baseline (speedup 1.0000x reference)
import jax
import jax.numpy as jnp
from jax.experimental import pallas as pl


_ROWS = 16384
_COLS = 2048
_BLK = 512


def _concat_kernel(left_ref, right_ref, out_ref):
    out_ref[:, :_COLS] = left_ref[:]
    out_ref[:, _COLS:] = right_ref[:]


def kernel(left, right):
    n_blk = _ROWS // _BLK
    return pl.pallas_call(
        _concat_kernel,
        grid=(n_blk,),
        in_specs=[
            pl.BlockSpec((_BLK, _COLS), lambda i: (i, 0)),
            pl.BlockSpec((_BLK, _COLS), lambda i: (i, 0)),
        ],
        out_specs=pl.BlockSpec((_BLK, 2 * _COLS), lambda i: (i, 0)),
        out_shape=jax.ShapeDtypeStruct((_ROWS, 2 * _COLS), jnp.float32),
    )(left, right)
